# baseline (speedup 1.0000x reference)
import functools

import jax
import jax.numpy as jnp
from jax import lax
from jax.experimental import pallas as pl
from jax.experimental.pallas import tpu as pltpu
from jax.experimental.pallas import tpu_sc as plsc

D = 768
N_PATCHES = 576
N_TOT = N_PATCHES + 1
BATCH = 128

NS = 16
LANES = 16


def _sc_cls_body(cls_hbm, pos_hbm, out_hbm, clsbuf, posbuf):
  wid = lax.axis_index("c") * NS + lax.axis_index("s")

  @pl.when(wid == 0)
  def _():
    pltpu.sync_copy(cls_hbm, clsbuf)
    pltpu.sync_copy(pos_hbm.at[pl.ds(0, D)], posbuf)
    for k in range(D // LANES):
      sl = pl.ds(k * LANES, LANES)
      clsbuf[sl] = clsbuf[sl] + posbuf[sl]
    pltpu.sync_copy(clsbuf, out_hbm)


TCB = 8


def _tc_body(cls_ref, x_ref, pos_ref, o_ref):
  o_ref[:, 0:1, :] = jnp.broadcast_to(cls_ref[...], (TCB, 1, D))
  o_ref[:, 1:, :] = x_ref[...] + pos_ref[1:, :]


@jax.jit
def kernel(inputs, class_embed, pos_table):
  mesh = plsc.VectorSubcoreMesh(core_axis_name="c", subcore_axis_name="s")
  sc_cls = functools.partial(
      pl.kernel,
      mesh=mesh,
      out_type=jax.ShapeDtypeStruct((D,), jnp.float32),
      scratch_types=[
          pltpu.VMEM((D,), jnp.float32),
          pltpu.VMEM((D,), jnp.float32),
      ],
  )(_sc_cls_body)
  cls_row = sc_cls(class_embed.reshape(-1), pos_table.reshape(-1))
  cls_row = cls_row.reshape(1, D)

  out = pl.pallas_call(
      _tc_body,
      grid=(BATCH // TCB,),
      in_specs=[
          pl.BlockSpec((1, D), lambda b: (0, 0)),
          pl.BlockSpec((TCB, N_PATCHES, D), lambda b: (b, 0, 0)),
          pl.BlockSpec((N_TOT, D), lambda b: (0, 0)),
      ],
      out_specs=pl.BlockSpec((TCB, N_TOT, D), lambda b: (b, 0, 0)),
      out_shape=jax.ShapeDtypeStruct((BATCH, N_TOT, D), jnp.float32),
  )(cls_row, inputs, pos_table)
  return out
